# baseline (speedup 1.0000x reference)
import functools

import jax
import jax.numpy as jnp
from jax import lax
from jax.experimental import pallas as pl
from jax.experimental.pallas import tpu as pltpu
from jax.experimental.pallas import tpu_sc as plsc

N_NODES = 10000
N_EDGES = 320000
D_IN = 128
HID = 8
HEADS = 8
HD = HEADS * HID

NT = 10112
ROW = HD + 16
TSW = HD // 2 + 16
ADW = 16
K = 128
NC, NS = 2, 16
NW = NC * NS
E_TOT = N_EDGES + N_NODES
NB = ((E_TOT + NW * K - 1) // (NW * K) + 1) // 2 * 2
EPW = NB * K
EPAD = NW * EPW
RPT = NT // NS

EROWS = N_EDGES // K
TROWS = (EPAD - N_EDGES) // K
BW = N_EDGES // EPW
BOFF = BW * NB
BSPLIT = EROWS - BOFF
BREM = NB - BSPLIT


def _dense_prologue(x, W1, asrc_flat, adst_flat):
    blk = 2000
    grid = N_NODES // blk

    def body(x_ref, w_ref, as_ref, ad_ref, hs_ref, ad_out_ref):
        h = jnp.dot(x_ref[...], w_ref[...], preferred_element_type=jnp.float32)
        pr = lax.broadcasted_iota(jnp.int32, (HD, HD), 0)
        pc = lax.broadcasted_iota(jnp.int32, (HD, HD), 1)
        perm = (pr == ((pc // 2) % HEADS) * HID + 2 * (pc // 16) + pc % 2
                ).astype(jnp.float32)
        h_perm = jnp.dot(h, perm, preferred_element_type=jnp.float32)
        qr = lax.broadcasted_iota(jnp.int32, (HD, 16), 0)
        qc = lax.broadcasted_iota(jnp.int32, (HD, 16), 1)
        PR = (qr // HID == qc % HEADS).astype(jnp.float32)
        a_s = jnp.dot(h * as_ref[...], PR, preferred_element_type=jnp.float32)
        a_d = jnp.dot(h * ad_ref[...], PR, preferred_element_type=jnp.float32)
        hs_ref[...] = jnp.concatenate([h_perm, a_s], axis=1)
        ad_out_ref[...] = a_d

    return pl.pallas_call(
        body,
        grid=(grid,),
        in_specs=[
            pl.BlockSpec((blk, D_IN), lambda i: (i, 0)),
            pl.BlockSpec((D_IN, HD), lambda i: (0, 0)),
            pl.BlockSpec((1, HD), lambda i: (0, 0)),
            pl.BlockSpec((1, HD), lambda i: (0, 0)),
        ],
        out_specs=[
            pl.BlockSpec((blk, ROW), lambda i: (i, 0)),
            pl.BlockSpec((blk, ADW), lambda i: (i, 0)),
        ],
        out_shape=[
            jax.ShapeDtypeStruct((NT, ROW), jnp.float32),
            jax.ShapeDtypeStruct((NT, ADW), jnp.float32),
        ],
    )(x, W1, asrc_flat, adst_flat)


def _sc_edge_pass(hs, ad, ei_rows, tail_rows):
    mesh = plsc.VectorSubcoreMesh(core_axis_name="c", subcore_axis_name="s")

    @functools.partial(
        pl.kernel,
        mesh=mesh,
        out_type=jax.ShapeDtypeStruct((NC, NT, ROW), jnp.float32),
        scratch_types=[
            pltpu.VMEM((NB, K), jnp.int32),
            pltpu.VMEM((NB, K), jnp.int32),
            pltpu.VMEM((2, K, TSW), jnp.float32),
            pltpu.VMEM((2, K, ADW), jnp.float32),
            pltpu.VMEM((2, K, ROW), jnp.float32),
            pltpu.VMEM_SHARED((NT, ROW), jnp.float32),
            pltpu.SemaphoreType.DMA,
            pltpu.SemaphoreType.DMA,
            pltpu.SemaphoreType.DMA,
            pltpu.SemaphoreType.DMA,
            pltpu.SemaphoreType.DMA,
            pltpu.SemaphoreType.DMA,
        ],
        compiler_params=pltpu.CompilerParams(use_tc_tiling_on_sc=False,
                                             needs_layout_passes=False),
    )
    def body(hs_hbm, ad_hbm, ei_hbm, tail_hbm, out_hbm,
             src_all, dst_all, S_v, D_v, W_v, acc, gs0, gs1, gd0, gd1,
             ss0, ss1):
        c = lax.axis_index("c")
        s = lax.axis_index("s")
        wid = s * NC + c
        r0 = s * RPT

        @plsc.parallel_loop(0, K)
        def zero_body(e):
            zv = jnp.zeros((16,), jnp.float32)
            for q in range(2):
                for v in range(5):
                    W_v[q, e, pl.ds(16 * v, 16)] = zv

        for q in range(4):
            pltpu.sync_copy(W_v.at[q % 2], acc.at[pl.ds(r0 + q * K, K)])
        pltpu.sync_copy(W_v.at[0, pl.ds(0, RPT - 4 * K)],
                        acc.at[pl.ds(r0 + 4 * K, RPT - 4 * K)])

        @pl.when(wid < BW)
        def _():
            pltpu.sync_copy(ei_hbm.at[0, pl.ds(wid * NB, NB)], src_all)
            pltpu.sync_copy(ei_hbm.at[1, pl.ds(wid * NB, NB)], dst_all)

        @pl.when(wid == BW)
        def _():
            pltpu.sync_copy(ei_hbm.at[0, pl.ds(BOFF, BSPLIT)],
                            src_all.at[pl.ds(0, BSPLIT)])
            pltpu.sync_copy(ei_hbm.at[1, pl.ds(BOFF, BSPLIT)],
                            dst_all.at[pl.ds(0, BSPLIT)])
            pltpu.sync_copy(tail_hbm.at[pl.ds(0, BREM)],
                            src_all.at[pl.ds(BSPLIT, BREM)])
            pltpu.sync_copy(tail_hbm.at[pl.ds(0, BREM)],
                            dst_all.at[pl.ds(BSPLIT, BREM)])

        @pl.when(wid == BW + 1)
        def _():
            pltpu.sync_copy(tail_hbm.at[pl.ds(BREM, NB)], src_all)
            pltpu.sync_copy(tail_hbm.at[pl.ds(BREM, NB)], dst_all)

        plsc.subcore_barrier()

        gsem = (gs0, gs1)
        gdem = (gd0, gd1)
        ssem = (ss0, ss1)

        def issue(j, r):
            pltpu.async_copy(hs_hbm.at[src_all.at[j]], S_v.at[r], gsem[r])
            pltpu.async_copy(ad_hbm.at[dst_all.at[j]], D_v.at[r], gdem[r])

        issue(0, 0)

        def pair_body(jj, carry):
            for b in range(2):
                j = 2 * jj + b
                r = b
                jn = jnp.minimum(j + 1, NB - 1)
                issue(jn, 1 - r)
                pltpu.make_async_copy(hs_hbm.at[src_all.at[j]],
                                      S_v.at[r], gsem[r]).wait()
                pltpu.make_async_copy(ad_hbm.at[dst_all.at[j]],
                                      D_v.at[r], gdem[r]).wait()

                @pl.when(jj > 0)
                def _():
                    pltpu.make_async_copy(
                        W_v.at[r], acc.at[dst_all.at[j]], ssem[r]).wait()

                @plsc.parallel_loop(0, K, unroll=8)
                def edge_body(e):
                    asv = S_v[r, e, pl.ds(HD // 2, 16)]
                    adv = D_v[r, e, pl.ds(0, 16)]
                    al = asv + adv
                    al = jnp.maximum(al, al * jnp.float32(0.2))
                    ex = jnp.exp(al)
                    W_v[r, e, pl.ds(HD, 16)] = ex
                    for g in range(2):
                        w = S_v[r, e, pl.ds(16 * g, 16)]
                        hw = plsc.bitcast(w, jnp.bfloat16)
                        he, ho = plsc.unpack(
                            hw, format=plsc.PackFormat.INTERLEAVED)
                        W_v[r, e, pl.ds(32 * g, 16)] = he * ex
                        W_v[r, e, pl.ds(32 * g + 16, 16)] = ho * ex

                pltpu.async_copy(W_v.at[r], acc.at[dst_all.at[j]],
                                 ssem[r], add=True)
            return carry

        lax.fori_loop(0, NB // 2, pair_body, 0)
        pltpu.make_async_copy(hs_hbm.at[src_all.at[NB - 1]],
                              S_v.at[0], gsem[0]).wait()
        pltpu.make_async_copy(ad_hbm.at[dst_all.at[NB - 1]],
                              D_v.at[0], gdem[0]).wait()
        pltpu.make_async_copy(W_v.at[0], acc.at[dst_all.at[NB - 2]],
                              ssem[0]).wait()
        pltpu.make_async_copy(W_v.at[1], acc.at[dst_all.at[NB - 1]],
                              ssem[1]).wait()
        plsc.subcore_barrier()
        pltpu.sync_copy(acc.at[pl.ds(r0, RPT)], out_hbm.at[c, pl.ds(r0, RPT)])

    return body(hs, ad, ei_rows, tail_rows)


def _epilogue(partials, b1_row):
    blk = 2000
    grid = N_NODES // blk

    def body(p_ref, b_ref, o_ref):
        acc = p_ref[0] + p_ref[1]
        num_p = acc[:, :HD]
        den16 = acc[:, HD:]
        tr = lax.broadcasted_iota(jnp.int32, (16, HD), 0)
        tc = lax.broadcasted_iota(jnp.int32, (16, HD), 1)
        T = (tr == tc % 16).astype(jnp.float32)
        den_p = jnp.dot(den16, T, preferred_element_type=jnp.float32)
        o_p = num_p / (den_p + jnp.float32(1e-16))
        ur = lax.broadcasted_iota(jnp.int32, (HD, HD), 0)
        uc = lax.broadcasted_iota(jnp.int32, (HD, HD), 1)
        ug = ur // 32
        ul = ur % 16
        uo = (ur // 16) % 2
        U = (uc == (ul % HEADS) * HID + 4 * ug + 2 * (ul // HEADS) + uo
             ).astype(jnp.float32)
        o = jnp.dot(o_p, U, preferred_element_type=jnp.float32) + b_ref[...]
        o_ref[...] = jnp.where(o > 0, o, jnp.exp(o) - jnp.float32(1.0))

    return pl.pallas_call(
        body,
        grid=(grid,),
        in_specs=[
            pl.BlockSpec((NC, blk, ROW), lambda i: (0, i, 0)),
            pl.BlockSpec((1, HD), lambda i: (0, 0)),
        ],
        out_specs=pl.BlockSpec((blk, HD), lambda i: (i, 0)),
        out_shape=jax.ShapeDtypeStruct((N_NODES, HD), jnp.float32),
    )(partials, b1_row)


def kernel(x, edge_index, edge_attr, W1, att_src1, att_dst1, b1,
           W2, att_src2, att_dst2, b2):
    del edge_attr, W2, att_src2, att_dst2, b2
    n = x.shape[0]
    asrc_flat = att_src1.reshape(1, HD)
    adst_flat = att_dst1.reshape(1, HD)

    hs_raw, ad = _dense_prologue(x, W1, asrc_flat, adst_flat)
    h_words = jax.lax.bitcast_convert_type(
        hs_raw[:, :HD].astype(jnp.bfloat16).reshape(NT, HD // 2, 2),
        jnp.float32)
    hs = jnp.concatenate([h_words, hs_raw[:, HD:]], axis=1)

    tail = jnp.concatenate([
        jnp.arange(n, dtype=jnp.int32),
        n + jnp.arange(EPAD - E_TOT, dtype=jnp.int32) % jnp.int32(NT - n),
    ]).reshape(TROWS, K)
    ei_rows = edge_index.reshape(2, EROWS, K)

    partials = _sc_edge_pass(hs, ad, ei_rows, tail)

    return _epilogue(partials, b1.reshape(1, HD))


# speedup vs baseline: 1.0185x; 1.0185x over previous
import functools

import jax
import jax.numpy as jnp
from jax import lax
from jax.experimental import pallas as pl
from jax.experimental.pallas import tpu as pltpu
from jax.experimental.pallas import tpu_sc as plsc

N_NODES = 10000
N_EDGES = 320000
D_IN = 128
HID = 8
HEADS = 8
HD = HEADS * HID

NT = 10112
ROW = HD + 16
ADW = 16
K = 128
NC, NS = 2, 16
NW = NC * NS
E_TOT = N_EDGES + N_NODES
NB = ((E_TOT + NW * K - 1) // (NW * K) + 1) // 2 * 2
EPW = NB * K
EPAD = NW * EPW
RPT = NT // NS

EROWS = N_EDGES // K
TROWS = (EPAD - N_EDGES) // K
BW = N_EDGES // EPW
BOFF = BW * NB
BSPLIT = EROWS - BOFF
BREM = NB - BSPLIT


def _dense_prologue(x, W1, asrc_flat, adst_flat):
    blk = 1000
    grid = N_NODES // blk

    def body(x_ref, w_ref, as_ref, ad_ref, hs_ref, ad_out_ref):
        h = jnp.dot(x_ref[...], w_ref[...], preferred_element_type=jnp.float32)
        pr = lax.broadcasted_iota(jnp.int32, (HD, HD), 0)
        pc = lax.broadcasted_iota(jnp.int32, (HD, HD), 1)
        perm = (pr == (pc % HEADS) * HID + pc // HEADS).astype(jnp.float32)
        h_perm = jnp.dot(h, perm, preferred_element_type=jnp.float32)
        qr = lax.broadcasted_iota(jnp.int32, (HD, 16), 0)
        qc = lax.broadcasted_iota(jnp.int32, (HD, 16), 1)
        PR = (qr // HID == qc % HEADS).astype(jnp.float32)
        a_s = jnp.dot(h * as_ref[...], PR, preferred_element_type=jnp.float32)
        a_d = jnp.dot(h * ad_ref[...], PR, preferred_element_type=jnp.float32)
        hs_ref[...] = jnp.concatenate([h_perm, a_s], axis=1)
        ad_out_ref[...] = a_d

    return pl.pallas_call(
        body,
        grid=(grid,),
        in_specs=[
            pl.BlockSpec((blk, D_IN), lambda i: (i, 0)),
            pl.BlockSpec((D_IN, HD), lambda i: (0, 0)),
            pl.BlockSpec((1, HD), lambda i: (0, 0)),
            pl.BlockSpec((1, HD), lambda i: (0, 0)),
        ],
        out_specs=[
            pl.BlockSpec((blk, ROW), lambda i: (i, 0)),
            pl.BlockSpec((blk, ADW), lambda i: (i, 0)),
        ],
        out_shape=[
            jax.ShapeDtypeStruct((NT, ROW), jnp.float32),
            jax.ShapeDtypeStruct((NT, ADW), jnp.float32),
        ],
    )(x, W1, asrc_flat, adst_flat)


def _sc_edge_pass(hs, ad, src_rows, dst_rows, tail_rows):
    mesh = plsc.VectorSubcoreMesh(core_axis_name="c", subcore_axis_name="s")

    @functools.partial(
        pl.kernel,
        mesh=mesh,
        out_type=jax.ShapeDtypeStruct((NC, NT, ROW), jnp.float32),
        scratch_types=[
            pltpu.VMEM((NB, K), jnp.int32),
            pltpu.VMEM((NB, K), jnp.int32),
            pltpu.VMEM((2, K, ROW), jnp.float32),
            pltpu.VMEM((2, K, ADW), jnp.float32),
            pltpu.VMEM((2, K, ROW), jnp.float32),
            pltpu.VMEM_SHARED((NT, ROW), jnp.float32),
            pltpu.SemaphoreType.DMA,
            pltpu.SemaphoreType.DMA,
            pltpu.SemaphoreType.DMA,
            pltpu.SemaphoreType.DMA,
            pltpu.SemaphoreType.DMA,
            pltpu.SemaphoreType.DMA,
        ],
        compiler_params=pltpu.CompilerParams(use_tc_tiling_on_sc=False),
    )
    def body(hs_hbm, ad_hbm, srce_hbm, dste_hbm, tail_hbm, out_hbm,
             src_all, dst_all, S_v, D_v, W_v, acc, gs0, gs1, gd0, gd1,
             ss0, ss1):
        c = lax.axis_index("c")
        s = lax.axis_index("s")
        wid = s * NC + c
        r0 = s * RPT

        @plsc.parallel_loop(0, K)
        def zero_body(e):
            zv = jnp.zeros((16,), jnp.float32)
            for q in range(2):
                for v in range(5):
                    W_v[q, e, pl.ds(16 * v, 16)] = zv

        for q in range(4):
            pltpu.sync_copy(W_v.at[q % 2], acc.at[pl.ds(r0 + q * K, K)])
        pltpu.sync_copy(W_v.at[0, pl.ds(0, RPT - 4 * K)],
                        acc.at[pl.ds(r0 + 4 * K, RPT - 4 * K)])

        @pl.when(wid < BW)
        def _():
            pltpu.sync_copy(srce_hbm.at[pl.ds(wid * NB, NB)], src_all)
            pltpu.sync_copy(dste_hbm.at[pl.ds(wid * NB, NB)], dst_all)

        @pl.when(wid == BW)
        def _():
            pltpu.sync_copy(srce_hbm.at[pl.ds(BOFF, BSPLIT)],
                            src_all.at[pl.ds(0, BSPLIT)])
            pltpu.sync_copy(dste_hbm.at[pl.ds(BOFF, BSPLIT)],
                            dst_all.at[pl.ds(0, BSPLIT)])
            pltpu.sync_copy(tail_hbm.at[pl.ds(0, BREM)],
                            src_all.at[pl.ds(BSPLIT, BREM)])
            pltpu.sync_copy(tail_hbm.at[pl.ds(0, BREM)],
                            dst_all.at[pl.ds(BSPLIT, BREM)])

        @pl.when(wid == BW + 1)
        def _():
            pltpu.sync_copy(tail_hbm.at[pl.ds(BREM, NB)], src_all)
            pltpu.sync_copy(tail_hbm.at[pl.ds(BREM, NB)], dst_all)

        plsc.subcore_barrier()

        gsem = (gs0, gs1)
        gdem = (gd0, gd1)
        ssem = (ss0, ss1)

        def issue(j, r):
            pltpu.async_copy(hs_hbm.at[src_all.at[j]], S_v.at[r], gsem[r])
            pltpu.async_copy(ad_hbm.at[dst_all.at[j]], D_v.at[r], gdem[r])

        issue(0, 0)

        def pair_body(jj, carry):
            for b in range(2):
                j = 2 * jj + b
                r = b
                jn = jnp.minimum(j + 1, NB - 1)
                issue(jn, 1 - r)
                pltpu.make_async_copy(hs_hbm.at[src_all.at[j]],
                                      S_v.at[r], gsem[r]).wait()
                pltpu.make_async_copy(ad_hbm.at[dst_all.at[j]],
                                      D_v.at[r], gdem[r]).wait()

                @pl.when(jj > 0)
                def _():
                    pltpu.make_async_copy(
                        W_v.at[r], acc.at[dst_all.at[j]], ssem[r]).wait()

                @plsc.parallel_loop(0, K, unroll=8)
                def edge_body(e):
                    asv = S_v[r, e, pl.ds(HD, 16)]
                    adv = D_v[r, e, pl.ds(0, 16)]
                    al = asv + adv
                    al = jnp.maximum(al, al * jnp.float32(0.2))
                    ex = jnp.exp(al)
                    W_v[r, e, pl.ds(HD, 16)] = ex
                    for v in range(4):
                        hv = S_v[r, e, pl.ds(16 * v, 16)]
                        W_v[r, e, pl.ds(16 * v, 16)] = hv * ex

                pltpu.async_copy(W_v.at[r], acc.at[dst_all.at[j]],
                                 ssem[r], add=True)
            return carry

        lax.fori_loop(0, NB // 2, pair_body, 0)
        pltpu.make_async_copy(hs_hbm.at[src_all.at[NB - 1]],
                              S_v.at[0], gsem[0]).wait()
        pltpu.make_async_copy(ad_hbm.at[dst_all.at[NB - 1]],
                              D_v.at[0], gdem[0]).wait()
        pltpu.make_async_copy(W_v.at[0], acc.at[dst_all.at[NB - 2]],
                              ssem[0]).wait()
        pltpu.make_async_copy(W_v.at[1], acc.at[dst_all.at[NB - 1]],
                              ssem[1]).wait()
        plsc.subcore_barrier()
        pltpu.sync_copy(acc.at[pl.ds(r0, RPT)], out_hbm.at[c, pl.ds(r0, RPT)])

    return body(hs, ad, src_rows, dst_rows, tail_rows)


def _epilogue(partials, b1_row):
    blk = 1000
    grid = N_NODES // blk

    def body(p_ref, b_ref, o_ref):
        acc = p_ref[0] + p_ref[1]
        num_p = acc[:, :HD]
        den16 = acc[:, HD:]
        tr = lax.broadcasted_iota(jnp.int32, (16, HD), 0)
        tc = lax.broadcasted_iota(jnp.int32, (16, HD), 1)
        T = (tr == tc % 16).astype(jnp.float32)
        den_p = jnp.dot(den16, T, preferred_element_type=jnp.float32)
        o_p = num_p / (den_p + jnp.float32(1e-16))
        ur = lax.broadcasted_iota(jnp.int32, (HD, HD), 0)
        uc = lax.broadcasted_iota(jnp.int32, (HD, HD), 1)
        U = (uc == (ur % HEADS) * HID + ur // HEADS).astype(jnp.float32)
        o = jnp.dot(o_p, U, preferred_element_type=jnp.float32) + b_ref[...]
        o_ref[...] = jnp.where(o > 0, o, jnp.exp(o) - jnp.float32(1.0))

    return pl.pallas_call(
        body,
        grid=(grid,),
        in_specs=[
            pl.BlockSpec((NC, blk, ROW), lambda i: (0, i, 0)),
            pl.BlockSpec((1, HD), lambda i: (0, 0)),
        ],
        out_specs=pl.BlockSpec((blk, HD), lambda i: (i, 0)),
        out_shape=jax.ShapeDtypeStruct((N_NODES, HD), jnp.float32),
    )(partials, b1_row)


def kernel(x, edge_index, edge_attr, W1, att_src1, att_dst1, b1,
           W2, att_src2, att_dst2, b2):
    del edge_attr, W2, att_src2, att_dst2, b2
    n = x.shape[0]
    asrc_flat = att_src1.reshape(1, HD)
    adst_flat = att_dst1.reshape(1, HD)

    hs, ad = _dense_prologue(x, W1, asrc_flat, adst_flat)

    tail = jnp.concatenate([
        jnp.arange(n, dtype=jnp.int32),
        n + jnp.arange(EPAD - E_TOT, dtype=jnp.int32) % jnp.int32(NT - n),
    ]).reshape(TROWS, K)
    src_rows = edge_index[0].reshape(EROWS, K)
    dst_rows = edge_index[1].reshape(EROWS, K)

    partials = _sc_edge_pass(hs, ad, src_rows, dst_rows, tail)

    return _epilogue(partials, b1.reshape(1, HD))


# speedup vs baseline: 1.0540x; 1.0348x over previous
import functools

import jax
import jax.numpy as jnp
from jax import lax
from jax.experimental import pallas as pl
from jax.experimental.pallas import tpu as pltpu
from jax.experimental.pallas import tpu_sc as plsc

N_NODES = 10000
N_EDGES = 320000
D_IN = 128
HID = 8
HEADS = 8
HD = HEADS * HID

NT = 10112
ROW = HD + 16
ADW = 16
K = 128
NC, NS = 2, 16
NW = NC * NS
E_TOT = N_EDGES + N_NODES
NB = ((E_TOT + NW * K - 1) // (NW * K) + 1) // 2 * 2
EPW = NB * K
EPAD = NW * EPW
RPT = NT // NS

EROWS = N_EDGES // K
TROWS = (EPAD - N_EDGES) // K
BW = N_EDGES // EPW
BOFF = BW * NB
BSPLIT = EROWS - BOFF
BREM = NB - BSPLIT


def _dense_prologue(x, W1, asrc_flat, adst_flat):
    blk = 2000
    grid = N_NODES // blk

    def body(x_ref, w_ref, as_ref, ad_ref, hs_ref, ad_out_ref):
        h = jnp.dot(x_ref[...], w_ref[...], preferred_element_type=jnp.float32)
        pr = lax.broadcasted_iota(jnp.int32, (HD, HD), 0)
        pc = lax.broadcasted_iota(jnp.int32, (HD, HD), 1)
        perm = (pr == (pc % HEADS) * HID + pc // HEADS).astype(jnp.float32)
        h_perm = jnp.dot(h, perm, preferred_element_type=jnp.float32)
        qr = lax.broadcasted_iota(jnp.int32, (HD, 16), 0)
        qc = lax.broadcasted_iota(jnp.int32, (HD, 16), 1)
        PR = (qr // HID == qc % HEADS).astype(jnp.float32)
        a_s = jnp.dot(h * as_ref[...], PR, preferred_element_type=jnp.float32)
        a_d = jnp.dot(h * ad_ref[...], PR, preferred_element_type=jnp.float32)
        hs_ref[...] = jnp.concatenate([h_perm, a_s], axis=1)
        ad_out_ref[...] = a_d

    return pl.pallas_call(
        body,
        grid=(grid,),
        in_specs=[
            pl.BlockSpec((blk, D_IN), lambda i: (i, 0)),
            pl.BlockSpec((D_IN, HD), lambda i: (0, 0)),
            pl.BlockSpec((1, HD), lambda i: (0, 0)),
            pl.BlockSpec((1, HD), lambda i: (0, 0)),
        ],
        out_specs=[
            pl.BlockSpec((blk, ROW), lambda i: (i, 0)),
            pl.BlockSpec((blk, ADW), lambda i: (i, 0)),
        ],
        out_shape=[
            jax.ShapeDtypeStruct((NT, ROW), jnp.float32),
            jax.ShapeDtypeStruct((NT, ADW), jnp.float32),
        ],
    )(x, W1, asrc_flat, adst_flat)


def _sc_edge_pass(hs, ad, src_rows, dst_rows, tail_rows):
    mesh = plsc.VectorSubcoreMesh(core_axis_name="c", subcore_axis_name="s")

    @functools.partial(
        pl.kernel,
        mesh=mesh,
        out_type=jax.ShapeDtypeStruct((NC, NT, ROW), jnp.float32),
        scratch_types=[
            pltpu.VMEM((NB, K), jnp.int32),
            pltpu.VMEM((NB, K), jnp.int32),
            pltpu.VMEM((2, K, ROW), jnp.float32),
            pltpu.VMEM((2, K, ADW), jnp.float32),
            pltpu.VMEM((2, K, ROW), jnp.float32),
            pltpu.VMEM_SHARED((NT, ROW), jnp.float32),
            pltpu.SemaphoreType.DMA,
            pltpu.SemaphoreType.DMA,
            pltpu.SemaphoreType.DMA,
            pltpu.SemaphoreType.DMA,
            pltpu.SemaphoreType.DMA,
            pltpu.SemaphoreType.DMA,
        ],
        compiler_params=pltpu.CompilerParams(use_tc_tiling_on_sc=False),
    )
    def body(hs_hbm, ad_hbm, srce_hbm, dste_hbm, tail_hbm, out_hbm,
             src_all, dst_all, S_v, D_v, W_v, acc, gs0, gs1, gd0, gd1,
             ss0, ss1):
        c = lax.axis_index("c")
        s = lax.axis_index("s")
        wid = s * NC + c
        r0 = s * RPT

        @plsc.parallel_loop(0, K)
        def zero_body(e):
            zv = jnp.zeros((16,), jnp.float32)
            for q in range(2):
                for v in range(5):
                    W_v[q, e, pl.ds(16 * v, 16)] = zv

        for q in range(4):
            pltpu.sync_copy(W_v.at[q % 2], acc.at[pl.ds(r0 + q * K, K)])
        pltpu.sync_copy(W_v.at[0, pl.ds(0, RPT - 4 * K)],
                        acc.at[pl.ds(r0 + 4 * K, RPT - 4 * K)])

        @pl.when(wid < BW)
        def _():
            pltpu.sync_copy(srce_hbm.at[pl.ds(wid * NB, NB)], src_all)
            pltpu.sync_copy(dste_hbm.at[pl.ds(wid * NB, NB)], dst_all)

        @pl.when(wid == BW)
        def _():
            pltpu.sync_copy(srce_hbm.at[pl.ds(BOFF, BSPLIT)],
                            src_all.at[pl.ds(0, BSPLIT)])
            pltpu.sync_copy(dste_hbm.at[pl.ds(BOFF, BSPLIT)],
                            dst_all.at[pl.ds(0, BSPLIT)])
            pltpu.sync_copy(tail_hbm.at[pl.ds(0, BREM)],
                            src_all.at[pl.ds(BSPLIT, BREM)])
            pltpu.sync_copy(tail_hbm.at[pl.ds(0, BREM)],
                            dst_all.at[pl.ds(BSPLIT, BREM)])

        @pl.when(wid == BW + 1)
        def _():
            pltpu.sync_copy(tail_hbm.at[pl.ds(BREM, NB)], src_all)
            pltpu.sync_copy(tail_hbm.at[pl.ds(BREM, NB)], dst_all)

        plsc.subcore_barrier()

        gsem = (gs0, gs1)
        gdem = (gd0, gd1)
        ssem = (ss0, ss1)

        def issue(j, r):
            pltpu.async_copy(hs_hbm.at[src_all.at[j]], S_v.at[r], gsem[r])
            pltpu.async_copy(ad_hbm.at[dst_all.at[j]], D_v.at[r], gdem[r])

        issue(0, 0)

        def pair_body(jj, carry):
            for b in range(2):
                j = 2 * jj + b
                r = b
                jn = jnp.minimum(j + 1, NB - 1)
                issue(jn, 1 - r)
                pltpu.make_async_copy(hs_hbm.at[src_all.at[j]],
                                      S_v.at[r], gsem[r]).wait()
                pltpu.make_async_copy(ad_hbm.at[dst_all.at[j]],
                                      D_v.at[r], gdem[r]).wait()

                @pl.when(jj > 0)
                def _():
                    pltpu.make_async_copy(
                        W_v.at[r], acc.at[dst_all.at[j]], ssem[r]).wait()

                @plsc.parallel_loop(0, K, unroll=8)
                def edge_body(e):
                    asv = S_v[r, e, pl.ds(HD, 16)]
                    adv = D_v[r, e, pl.ds(0, 16)]
                    al = asv + adv
                    al = jnp.maximum(al, al * jnp.float32(0.2))
                    ex = jnp.exp(al)
                    W_v[r, e, pl.ds(HD, 16)] = ex
                    for v in range(4):
                        hv = S_v[r, e, pl.ds(16 * v, 16)]
                        W_v[r, e, pl.ds(16 * v, 16)] = hv * ex

                pltpu.async_copy(W_v.at[r], acc.at[dst_all.at[j]],
                                 ssem[r], add=True)
            return carry

        lax.fori_loop(0, NB // 2, pair_body, 0)
        pltpu.make_async_copy(hs_hbm.at[src_all.at[NB - 1]],
                              S_v.at[0], gsem[0]).wait()
        pltpu.make_async_copy(ad_hbm.at[dst_all.at[NB - 1]],
                              D_v.at[0], gdem[0]).wait()
        pltpu.make_async_copy(W_v.at[0], acc.at[dst_all.at[NB - 2]],
                              ssem[0]).wait()
        pltpu.make_async_copy(W_v.at[1], acc.at[dst_all.at[NB - 1]],
                              ssem[1]).wait()
        plsc.subcore_barrier()
        pltpu.sync_copy(acc.at[pl.ds(r0, RPT)], out_hbm.at[c, pl.ds(r0, RPT)])

    return body(hs, ad, src_rows, dst_rows, tail_rows)


def _epilogue(partials, b1_row):
    blk = 2000
    grid = N_NODES // blk

    def body(p_ref, b_ref, o_ref):
        acc = p_ref[0] + p_ref[1]
        num_p = acc[:, :HD]
        den16 = acc[:, HD:]
        tr = lax.broadcasted_iota(jnp.int32, (16, HD), 0)
        tc = lax.broadcasted_iota(jnp.int32, (16, HD), 1)
        T = (tr == tc % 16).astype(jnp.float32)
        den_p = jnp.dot(den16, T, preferred_element_type=jnp.float32)
        o_p = num_p / (den_p + jnp.float32(1e-16))
        ur = lax.broadcasted_iota(jnp.int32, (HD, HD), 0)
        uc = lax.broadcasted_iota(jnp.int32, (HD, HD), 1)
        U = (uc == (ur % HEADS) * HID + ur // HEADS).astype(jnp.float32)
        o = jnp.dot(o_p, U, preferred_element_type=jnp.float32) + b_ref[...]
        o_ref[...] = jnp.where(o > 0, o, jnp.exp(o) - jnp.float32(1.0))

    return pl.pallas_call(
        body,
        grid=(grid,),
        in_specs=[
            pl.BlockSpec((NC, blk, ROW), lambda i: (0, i, 0)),
            pl.BlockSpec((1, HD), lambda i: (0, 0)),
        ],
        out_specs=pl.BlockSpec((blk, HD), lambda i: (i, 0)),
        out_shape=jax.ShapeDtypeStruct((N_NODES, HD), jnp.float32),
    )(partials, b1_row)


def kernel(x, edge_index, edge_attr, W1, att_src1, att_dst1, b1,
           W2, att_src2, att_dst2, b2):
    del edge_attr, W2, att_src2, att_dst2, b2
    n = x.shape[0]
    asrc_flat = att_src1.reshape(1, HD)
    adst_flat = att_dst1.reshape(1, HD)

    hs, ad = _dense_prologue(x, W1, asrc_flat, adst_flat)

    tail = jnp.concatenate([
        jnp.arange(n, dtype=jnp.int32),
        n + jnp.arange(EPAD - E_TOT, dtype=jnp.int32) % jnp.int32(NT - n),
    ]).reshape(TROWS, K)
    src_rows = edge_index[0].reshape(EROWS, K)
    dst_rows = edge_index[1].reshape(EROWS, K)

    partials = _sc_edge_pass(hs, ad, src_rows, dst_rows, tail)

    return _epilogue(partials, b1.reshape(1, HD))


# speedup vs baseline: 1.1201x; 1.0627x over previous
import functools

import jax
import jax.numpy as jnp
from jax import lax
from jax.experimental import pallas as pl
from jax.experimental.pallas import tpu as pltpu
from jax.experimental.pallas import tpu_sc as plsc

N_NODES = 10000
N_EDGES = 320000
D_IN = 128
HID = 8
HEADS = 8
HD = HEADS * HID

NT = 10112
ROW = HD + 16
ADW = 16
K = 128
NC, NS = 2, 16
NW = NC * NS
E_TOT = N_EDGES + N_NODES
NB = ((E_TOT + NW * K - 1) // (NW * K) + 1) // 2 * 2
EPW = NB * K
EPAD = NW * EPW
RPT = NT // NS

EROWS = N_EDGES // K
TROWS = (EPAD - N_EDGES) // K
BW = N_EDGES // EPW
BOFF = BW * NB
BSPLIT = EROWS - BOFF
BREM = NB - BSPLIT


def _dense_prologue(x, W1, asrc_flat, adst_flat):
    blk = 2000
    grid = N_NODES // blk

    def body(x_ref, w_ref, as_ref, ad_ref, hs_ref, ad_out_ref):
        h = jnp.dot(x_ref[...], w_ref[...], preferred_element_type=jnp.float32)
        pr = lax.broadcasted_iota(jnp.int32, (HD, HD), 0)
        pc = lax.broadcasted_iota(jnp.int32, (HD, HD), 1)
        perm = (pr == (pc % HEADS) * HID + pc // HEADS).astype(jnp.float32)
        h_perm = jnp.dot(h, perm, preferred_element_type=jnp.float32)
        qr = lax.broadcasted_iota(jnp.int32, (HD, 16), 0)
        qc = lax.broadcasted_iota(jnp.int32, (HD, 16), 1)
        PR = (qr // HID == qc % HEADS).astype(jnp.float32)
        a_s = jnp.dot(h * as_ref[...], PR, preferred_element_type=jnp.float32)
        a_d = jnp.dot(h * ad_ref[...], PR, preferred_element_type=jnp.float32)
        hs_ref[...] = jnp.concatenate([h_perm, a_s], axis=1)
        ad_out_ref[...] = a_d

    return pl.pallas_call(
        body,
        grid=(grid,),
        in_specs=[
            pl.BlockSpec((blk, D_IN), lambda i: (i, 0)),
            pl.BlockSpec((D_IN, HD), lambda i: (0, 0)),
            pl.BlockSpec((1, HD), lambda i: (0, 0)),
            pl.BlockSpec((1, HD), lambda i: (0, 0)),
        ],
        out_specs=[
            pl.BlockSpec((blk, ROW), lambda i: (i, 0)),
            pl.BlockSpec((blk, ADW), lambda i: (i, 0)),
        ],
        out_shape=[
            jax.ShapeDtypeStruct((NT, ROW), jnp.float32),
            jax.ShapeDtypeStruct((NT, ADW), jnp.float32),
        ],
    )(x, W1, asrc_flat, adst_flat)


def _sc_edge_pass(hs, ad, ei_rows, tail_rows):
    mesh = plsc.VectorSubcoreMesh(core_axis_name="c", subcore_axis_name="s")

    @functools.partial(
        pl.kernel,
        mesh=mesh,
        out_type=jax.ShapeDtypeStruct((NC, NT, ROW), jnp.float32),
        scratch_types=[
            pltpu.VMEM((NB, K), jnp.int32),
            pltpu.VMEM((NB, K), jnp.int32),
            pltpu.VMEM((2, K, ROW), jnp.float32),
            pltpu.VMEM((2, K, ADW), jnp.float32),
            pltpu.VMEM((2, K, ROW), jnp.float32),
            pltpu.VMEM_SHARED((NT, ROW), jnp.float32),
            pltpu.SemaphoreType.DMA,
            pltpu.SemaphoreType.DMA,
            pltpu.SemaphoreType.DMA,
            pltpu.SemaphoreType.DMA,
            pltpu.SemaphoreType.DMA,
            pltpu.SemaphoreType.DMA,
        ],
        compiler_params=pltpu.CompilerParams(use_tc_tiling_on_sc=False),
    )
    def body(hs_hbm, ad_hbm, ei_hbm, tail_hbm, out_hbm,
             src_all, dst_all, S_v, D_v, W_v, acc, gs0, gs1, gd0, gd1,
             ss0, ss1):
        c = lax.axis_index("c")
        s = lax.axis_index("s")
        wid = s * NC + c
        r0 = s * RPT

        @plsc.parallel_loop(0, K)
        def zero_body(e):
            zv = jnp.zeros((16,), jnp.float32)
            for q in range(2):
                for v in range(5):
                    W_v[q, e, pl.ds(16 * v, 16)] = zv

        for q in range(4):
            pltpu.sync_copy(W_v.at[q % 2], acc.at[pl.ds(r0 + q * K, K)])
        pltpu.sync_copy(W_v.at[0, pl.ds(0, RPT - 4 * K)],
                        acc.at[pl.ds(r0 + 4 * K, RPT - 4 * K)])

        @pl.when(wid < BW)
        def _():
            pltpu.sync_copy(ei_hbm.at[0, pl.ds(wid * NB, NB)], src_all)
            pltpu.sync_copy(ei_hbm.at[1, pl.ds(wid * NB, NB)], dst_all)

        @pl.when(wid == BW)
        def _():
            pltpu.sync_copy(ei_hbm.at[0, pl.ds(BOFF, BSPLIT)],
                            src_all.at[pl.ds(0, BSPLIT)])
            pltpu.sync_copy(ei_hbm.at[1, pl.ds(BOFF, BSPLIT)],
                            dst_all.at[pl.ds(0, BSPLIT)])
            pltpu.sync_copy(tail_hbm.at[pl.ds(0, BREM)],
                            src_all.at[pl.ds(BSPLIT, BREM)])
            pltpu.sync_copy(tail_hbm.at[pl.ds(0, BREM)],
                            dst_all.at[pl.ds(BSPLIT, BREM)])

        @pl.when(wid == BW + 1)
        def _():
            pltpu.sync_copy(tail_hbm.at[pl.ds(BREM, NB)], src_all)
            pltpu.sync_copy(tail_hbm.at[pl.ds(BREM, NB)], dst_all)

        plsc.subcore_barrier()

        gsem = (gs0, gs1)
        gdem = (gd0, gd1)
        ssem = (ss0, ss1)

        def issue(j, r):
            pltpu.async_copy(hs_hbm.at[src_all.at[j]], S_v.at[r], gsem[r])
            pltpu.async_copy(ad_hbm.at[dst_all.at[j]], D_v.at[r], gdem[r])

        issue(0, 0)

        def pair_body(jj, carry):
            for b in range(2):
                j = 2 * jj + b
                r = b
                jn = jnp.minimum(j + 1, NB - 1)
                issue(jn, 1 - r)
                pltpu.make_async_copy(hs_hbm.at[src_all.at[j]],
                                      S_v.at[r], gsem[r]).wait()
                pltpu.make_async_copy(ad_hbm.at[dst_all.at[j]],
                                      D_v.at[r], gdem[r]).wait()

                @pl.when(jj > 0)
                def _():
                    pltpu.make_async_copy(
                        W_v.at[r], acc.at[dst_all.at[j]], ssem[r]).wait()

                @plsc.parallel_loop(0, K, unroll=8)
                def edge_body(e):
                    asv = S_v[r, e, pl.ds(HD, 16)]
                    adv = D_v[r, e, pl.ds(0, 16)]
                    al = asv + adv
                    al = jnp.maximum(al, al * jnp.float32(0.2))
                    ex = jnp.exp(al)
                    W_v[r, e, pl.ds(HD, 16)] = ex
                    for v in range(4):
                        hv = S_v[r, e, pl.ds(16 * v, 16)]
                        W_v[r, e, pl.ds(16 * v, 16)] = hv * ex

                pltpu.async_copy(W_v.at[r], acc.at[dst_all.at[j]],
                                 ssem[r], add=True)
            return carry

        lax.fori_loop(0, NB // 2, pair_body, 0)
        pltpu.make_async_copy(hs_hbm.at[src_all.at[NB - 1]],
                              S_v.at[0], gsem[0]).wait()
        pltpu.make_async_copy(ad_hbm.at[dst_all.at[NB - 1]],
                              D_v.at[0], gdem[0]).wait()
        pltpu.make_async_copy(W_v.at[0], acc.at[dst_all.at[NB - 2]],
                              ssem[0]).wait()
        pltpu.make_async_copy(W_v.at[1], acc.at[dst_all.at[NB - 1]],
                              ssem[1]).wait()
        plsc.subcore_barrier()
        pltpu.sync_copy(acc.at[pl.ds(r0, RPT)], out_hbm.at[c, pl.ds(r0, RPT)])

    return body(hs, ad, ei_rows, tail_rows)


def _epilogue(partials, b1_row):
    blk = 2000
    grid = N_NODES // blk

    def body(p_ref, b_ref, o_ref):
        acc = p_ref[0] + p_ref[1]
        num_p = acc[:, :HD]
        den16 = acc[:, HD:]
        tr = lax.broadcasted_iota(jnp.int32, (16, HD), 0)
        tc = lax.broadcasted_iota(jnp.int32, (16, HD), 1)
        T = (tr == tc % 16).astype(jnp.float32)
        den_p = jnp.dot(den16, T, preferred_element_type=jnp.float32)
        o_p = num_p / (den_p + jnp.float32(1e-16))
        ur = lax.broadcasted_iota(jnp.int32, (HD, HD), 0)
        uc = lax.broadcasted_iota(jnp.int32, (HD, HD), 1)
        U = (uc == (ur % HEADS) * HID + ur // HEADS).astype(jnp.float32)
        o = jnp.dot(o_p, U, preferred_element_type=jnp.float32) + b_ref[...]
        o_ref[...] = jnp.where(o > 0, o, jnp.exp(o) - jnp.float32(1.0))

    return pl.pallas_call(
        body,
        grid=(grid,),
        in_specs=[
            pl.BlockSpec((NC, blk, ROW), lambda i: (0, i, 0)),
            pl.BlockSpec((1, HD), lambda i: (0, 0)),
        ],
        out_specs=pl.BlockSpec((blk, HD), lambda i: (i, 0)),
        out_shape=jax.ShapeDtypeStruct((N_NODES, HD), jnp.float32),
    )(partials, b1_row)


def kernel(x, edge_index, edge_attr, W1, att_src1, att_dst1, b1,
           W2, att_src2, att_dst2, b2):
    del edge_attr, W2, att_src2, att_dst2, b2
    n = x.shape[0]
    asrc_flat = att_src1.reshape(1, HD)
    adst_flat = att_dst1.reshape(1, HD)

    hs, ad = _dense_prologue(x, W1, asrc_flat, adst_flat)

    tail = jnp.concatenate([
        jnp.arange(n, dtype=jnp.int32),
        n + jnp.arange(EPAD - E_TOT, dtype=jnp.int32) % jnp.int32(NT - n),
    ]).reshape(TROWS, K)
    ei_rows = edge_index.reshape(2, EROWS, K)

    partials = _sc_edge_pass(hs, ad, ei_rows, tail)

    return _epilogue(partials, b1.reshape(1, HD))
